# Initial kernel scaffold; baseline (speedup 1.0000x reference)
import jax
import jax.numpy as jnp
from jax.experimental import pallas as pl


def kernel(codes, tables):
    raise NotImplementedError("write your pallas kernel here")


# speedup vs baseline: 2.0627x; 2.0627x over previous
import functools

import jax
import jax.numpy as jnp
from jax import lax
from jax.experimental import pallas as pl
from jax.experimental.pallas import tpu as pltpu
from jax.experimental.pallas import tpu_sc as plsc

B = 4
C = 8
T = 4096
V = 2048
D = 1024

NROWS = B * T
NW = 32
RPW = NROWS // NW
R = 8
NCH = RPW // R
G = R * C
NL = 16


def _sc_embed(codes_flat, tables_flat):
    mesh = plsc.VectorSubcoreMesh(core_axis_name="c", subcore_axis_name="s")

    @functools.partial(
        pl.kernel,
        mesh=mesh,
        out_type=jax.ShapeDtypeStruct((NROWS, D), jnp.float32),
        scratch_types=[
            pltpu.VMEM((G,), jnp.int32),
            pltpu.VMEM((G, D), jnp.float32),
            pltpu.VMEM((R, D), jnp.float32),
            pltpu.SemaphoreType.DMA,
        ],
    )
    def k(codes_hbm, tab_hbm, out_hbm, cbuf, gbuf, obuf, sem):
        wid = lax.axis_index("s") * 2 + lax.axis_index("c")
        base = wid * RPW
        lane = lax.iota(jnp.int32, NL)
        offpat = (lane & (C - 1)) * V

        def chunk(ci, _):
            row0 = base + ci * R
            pltpu.sync_copy(codes_hbm.at[pl.ds(row0 * C, G)], cbuf)
            for g in range(G // NL):
                sl = pl.ds(g * NL, NL)
                cbuf[sl] = cbuf[sl] + offpat
            pltpu.async_copy(tab_hbm.at[cbuf], gbuf, sem).wait()

            def reduce_group(g, _):
                sl = pl.ds(g * NL, NL)
                for r in range(R):
                    acc = gbuf[r * C, sl]
                    for i in range(1, C):
                        acc = acc + gbuf[r * C + i, sl]
                    obuf[r, sl] = acc
                return 0

            lax.fori_loop(0, D // NL, reduce_group, 0)
            pltpu.sync_copy(obuf, out_hbm.at[pl.ds(row0, R)])
            return 0

        lax.fori_loop(0, NCH, chunk, 0)

    return k(codes_flat, tables_flat)


def kernel(codes, tables):
    codes_flat = codes.transpose(0, 2, 1).reshape(-1)
    tables_flat = tables.reshape(C * V, D)
    out = _sc_embed(codes_flat, tables_flat)
    return out.reshape(B, T, D)
